# baseline (speedup 1.0000x reference)
import jax
import jax.numpy as jnp
from jax.experimental import pallas as pl
from jax.experimental.pallas import tpu as pltpu


def _add_kernel(x_ref, e_ref, o_ref):
    o_ref[...] = x_ref[...] + e_ref[...][None, :, None, :]


def kernel(x, embedding):
    B, N, K, D = x.shape
    n_blk = 256
    grid = (B, N // n_blk)
    return pl.pallas_call(
        _add_kernel,
        grid=grid,
        in_specs=[
            pl.BlockSpec((1, n_blk, K, D), lambda b, j: (b, j, 0, 0)),
            pl.BlockSpec((n_blk, D), lambda b, j: (j, 0)),
        ],
        out_specs=pl.BlockSpec((1, n_blk, K, D), lambda b, j: (b, j, 0, 0)),
        out_shape=jax.ShapeDtypeStruct(x.shape, x.dtype),
        compiler_params=pltpu.CompilerParams(
            vmem_limit_bytes=60 * 1024 * 1024,
        ),
    )(x, embedding)


# speedup vs baseline: 1.0823x; 1.0823x over previous
import jax
import jax.numpy as jnp
from jax import lax
from jax.experimental import pallas as pl
from jax.experimental.pallas import tpu as pltpu


def _make_body(B, N, K, D, NBUF, n_c):
    n_per_b = N // n_c
    total = B * n_per_b

    def body(x_hbm, e_ref, o_hbm, ibufs, obufs, in_sems, out_sems):
        def start_in(c, slot, b, j):
            pltpu.make_async_copy(
                x_hbm.at[b, pl.ds(j * n_c, n_c)],
                ibufs.at[slot],
                in_sems.at[slot],
            ).start()

        for c in range(min(NBUF, total)):
            start_in(c, c, c // n_per_b, c % n_per_b)

        def step(c, _):
            slot = lax.rem(c, NBUF)
            b = lax.div(c, n_per_b)
            j = lax.rem(c, n_per_b)

            pltpu.make_async_copy(
                x_hbm.at[b, pl.ds(j * n_c, n_c)],
                ibufs.at[slot],
                in_sems.at[slot],
            ).wait()

            @pl.when(c >= NBUF)
            def _():
                pltpu.make_async_copy(
                    obufs.at[slot],
                    o_hbm.at[b, pl.ds(j * n_c, n_c)],
                    out_sems.at[slot],
                ).wait()

            e_blk = e_ref[pl.ds(j * n_c, n_c), :]
            obufs[slot] = ibufs[slot] + e_blk[:, None, :]

            pltpu.make_async_copy(
                obufs.at[slot],
                o_hbm.at[b, pl.ds(j * n_c, n_c)],
                out_sems.at[slot],
            ).start()

            @pl.when(c + NBUF < total)
            def _():
                c2 = c + NBUF
                b2 = lax.div(c2, n_per_b)
                j2 = lax.rem(c2, n_per_b)
                pltpu.make_async_copy(
                    x_hbm.at[b2, pl.ds(j2 * n_c, n_c)],
                    ibufs.at[slot],
                    in_sems.at[slot],
                ).start()

            return 0

        lax.fori_loop(0, total, step, 0)

        for c in range(max(0, total - NBUF), total):
            slot = c % NBUF
            b, j = c // n_per_b, c % n_per_b
            pltpu.make_async_copy(
                obufs.at[slot],
                o_hbm.at[b, pl.ds(j * n_c, n_c)],
                out_sems.at[slot],
            ).wait()

    return body


def kernel(x, embedding):
    B, N, K, D = x.shape
    NBUF = 8
    n_c = 128
    body = _make_body(B, N, K, D, NBUF, n_c)
    return pl.pallas_call(
        body,
        grid=(),
        in_specs=[
            pl.BlockSpec(memory_space=pl.ANY),
            pl.BlockSpec(memory_space=pltpu.MemorySpace.VMEM),
        ],
        out_specs=pl.BlockSpec(memory_space=pl.ANY),
        out_shape=jax.ShapeDtypeStruct(x.shape, x.dtype),
        scratch_shapes=[
            pltpu.VMEM((NBUF, n_c, K, D), jnp.float32),
            pltpu.VMEM((NBUF, n_c, K, D), jnp.float32),
            pltpu.SemaphoreType.DMA((NBUF,)),
            pltpu.SemaphoreType.DMA((NBUF,)),
        ],
        compiler_params=pltpu.CompilerParams(
            vmem_limit_bytes=100 * 1024 * 1024,
        ),
    )(x, embedding)
